# Initial kernel scaffold; baseline (speedup 1.0000x reference)
import jax
import jax.numpy as jnp
from jax.experimental import pallas as pl


def kernel(x, edge_index, W1, b1, Wm1, bm1, Wm2, bm2, Wm3, bm3):
    raise NotImplementedError("write your pallas kernel here")


# speedup vs baseline: 10.4336x; 10.4336x over previous
import functools

import jax
import jax.numpy as jnp
from jax import lax
from jax.experimental import pallas as pl
from jax.experimental.pallas import tpu as pltpu
from jax.experimental.pallas import tpu_sc as plsc

N_NODES = 10000
N_EDGES = 320000
D_FEAT = 128
HIDDEN = 128
NUM_CLASSES = 10

NC = 2
NS = 16
CHUNK = 128
N_PAD = 10240
E_PAD = 327680
ROWS_PER_TILE = N_PAD // NS

_mesh = plsc.VectorSubcoreMesh(core_axis_name="c", subcore_axis_name="s")


def _deg_body(dst_hbm, zeros_hbm, ones_hbm, out_hbm, acc, idx_v, ones_v):
    c = lax.axis_index("c")
    s = lax.axis_index("s")
    r0 = s * ROWS_PER_TILE
    pltpu.sync_copy(zeros_hbm.at[pl.ds(r0, ROWS_PER_TILE)],
                    acc.at[pl.ds(r0, ROWS_PER_TILE)])
    pltpu.sync_copy(ones_hbm, ones_v)
    plsc.subcore_barrier()
    tile = s * NC + c
    n_chunks = E_PAD // (NC * NS) // CHUNK
    base0 = tile * (n_chunks * CHUNK)

    def step(i, carry):
        base = base0 + i * CHUNK
        pltpu.sync_copy(dst_hbm.at[pl.ds(base, CHUNK)], idx_v)
        pltpu.sync_copy(ones_v, acc.at[idx_v], add=True)
        return carry

    lax.fori_loop(0, n_chunks, step, 0)
    plsc.subcore_barrier()
    pltpu.sync_copy(acc.at[pl.ds(r0, ROWS_PER_TILE)],
                    out_hbm.at[c, pl.ds(r0, ROWS_PER_TILE)])


_deg_kernel = functools.partial(
    pl.kernel,
    out_type=jax.ShapeDtypeStruct((NC, N_PAD, 16), jnp.float32),
    mesh=_mesh,
    scratch_types=[
        pltpu.VMEM_SHARED((N_PAD, 16), jnp.float32),
        pltpu.VMEM((CHUNK,), jnp.int32),
        pltpu.VMEM((CHUNK, 16), jnp.float32),
    ],
)(_deg_body)


def _agg_body(src_hbm, dst_hbm, hp_hbm, zeros_hbm, out_hbm,
              acc, idxs_v, idxd_v, rows_v):
    c = lax.axis_index("c")
    s = lax.axis_index("s")
    r0 = s * ROWS_PER_TILE
    pltpu.sync_copy(zeros_hbm.at[pl.ds(r0, ROWS_PER_TILE)],
                    acc.at[pl.ds(r0, ROWS_PER_TILE)])
    plsc.subcore_barrier()
    tile = s * NC + c
    n_chunks = E_PAD // (NC * NS) // CHUNK
    base0 = tile * (n_chunks * CHUNK)

    def step(i, carry):
        base = base0 + i * CHUNK
        pltpu.sync_copy(src_hbm.at[pl.ds(base, CHUNK)], idxs_v)
        pltpu.sync_copy(dst_hbm.at[pl.ds(base, CHUNK)], idxd_v)
        pltpu.sync_copy(hp_hbm.at[idxs_v], rows_v)
        pltpu.sync_copy(rows_v, acc.at[idxd_v], add=True)
        return carry

    lax.fori_loop(0, n_chunks, step, 0)
    plsc.subcore_barrier()
    pltpu.sync_copy(acc.at[pl.ds(r0, ROWS_PER_TILE)],
                    out_hbm.at[c, pl.ds(r0, ROWS_PER_TILE)])


_agg_kernel = functools.partial(
    pl.kernel,
    out_type=jax.ShapeDtypeStruct((NC, N_PAD, D_FEAT), jnp.float32),
    mesh=_mesh,
    scratch_types=[
        pltpu.VMEM_SHARED((N_PAD, D_FEAT), jnp.float32),
        pltpu.VMEM((CHUNK,), jnp.int32),
        pltpu.VMEM((CHUNK,), jnp.int32),
        pltpu.VMEM((CHUNK, D_FEAT), jnp.float32),
    ],
)(_agg_body)


ROW_BLK = 400


def _hprime_body(x_ref, w_ref, deg_ref, hp_ref):
    d = deg_ref[0, :, 0:1] + deg_ref[1, :, 0:1] + 1.0
    dis = lax.rsqrt(d)
    h = jnp.dot(x_ref[...], w_ref[...], preferred_element_type=jnp.float32)
    hp_ref[...] = dis * h


def _hprime(x, w1, deg2):
    return pl.pallas_call(
        _hprime_body,
        grid=(N_NODES // ROW_BLK,),
        in_specs=[
            pl.BlockSpec((ROW_BLK, D_FEAT), lambda r: (r, 0)),
            pl.BlockSpec((D_FEAT, HIDDEN), lambda r: (0, 0)),
            pl.BlockSpec((NC, ROW_BLK, 16), lambda r: (0, r, 0)),
        ],
        out_specs=pl.BlockSpec((ROW_BLK, HIDDEN), lambda r: (r, 0)),
        out_shape=jax.ShapeDtypeStruct((N_NODES, HIDDEN), jnp.float32),
    )(x, w1, deg2)


NC_PAD = 128


def _head_body(acc_ref, hp_ref, deg_ref, b1_ref, wm1_ref, bm1_ref,
               wm2_ref, bm2_ref, wm3_ref, bm3_ref, out_ref):
    d = deg_ref[0, :, 0:1] + deg_ref[1, :, 0:1] + 1.0
    dis = lax.rsqrt(d)
    accf = acc_ref[0] + acc_ref[1]
    g = dis * (accf + hp_ref[...]) + b1_ref[...]
    h1 = jnp.maximum(g, 0.0)
    h2 = jnp.maximum(
        jnp.dot(h1, wm1_ref[...], preferred_element_type=jnp.float32)
        + bm1_ref[...], 0.0)
    h3 = jnp.maximum(
        jnp.dot(h2, wm2_ref[...], preferred_element_type=jnp.float32)
        + bm2_ref[...], 0.0)
    logits = (jnp.dot(h3, wm3_ref[...], preferred_element_type=jnp.float32)
              + bm3_ref[...])
    m = jnp.max(logits, axis=1, keepdims=True)
    lse = jnp.log(jnp.sum(jnp.exp(logits - m), axis=1, keepdims=True)) + m
    out_ref[...] = logits - lse


def _head(acc2, hp, deg2, b1, wm1, bm1, wm2, bm2, wm3p, bm3p):
    full = lambda shape: pl.BlockSpec(shape, lambda r: tuple(0 for _ in shape))
    return pl.pallas_call(
        _head_body,
        grid=(N_NODES // ROW_BLK,),
        in_specs=[
            pl.BlockSpec((NC, ROW_BLK, D_FEAT), lambda r: (0, r, 0)),
            pl.BlockSpec((ROW_BLK, HIDDEN), lambda r: (r, 0)),
            pl.BlockSpec((NC, ROW_BLK, 16), lambda r: (0, r, 0)),
            full((1, HIDDEN)),
            full((HIDDEN, HIDDEN // 2)),
            full((1, HIDDEN // 2)),
            full((HIDDEN // 2, HIDDEN // 4)),
            full((1, HIDDEN // 4)),
            full((HIDDEN // 4, NC_PAD)),
            full((1, NC_PAD)),
        ],
        out_specs=pl.BlockSpec((ROW_BLK, NC_PAD), lambda r: (r, 0)),
        out_shape=jax.ShapeDtypeStruct((N_NODES, NC_PAD), jnp.float32),
    )(acc2, hp, deg2, b1, wm1, bm1, wm2, bm2, wm3p, bm3p)


@jax.jit
def kernel(x, edge_index, W1, b1, Wm1, bm1, Wm2, bm2, Wm3, bm3):
    src = edge_index[0]
    dst = edge_index[1]
    pad = E_PAD - N_EDGES
    src_p = jnp.concatenate([src, jnp.zeros((pad,), jnp.int32)])
    dst_p = jnp.concatenate([dst, jnp.full((pad,), N_NODES, jnp.int32)])

    zeros16 = jnp.zeros((N_PAD, 16), jnp.float32)
    ones16 = jnp.ones((CHUNK, 16), jnp.float32)
    zeros128 = jnp.zeros((N_PAD, D_FEAT), jnp.float32)

    deg2 = _deg_kernel(dst_p, zeros16, ones16)
    hp = _hprime(x, W1, deg2)
    acc2 = _agg_kernel(src_p, dst_p, hp, zeros128)

    wm3p = jnp.zeros((HIDDEN // 4, NC_PAD), jnp.float32).at[:, :NUM_CLASSES].set(Wm3)
    bm3p = jnp.full((NC_PAD,), -1e30, jnp.float32).at[:NUM_CLASSES].set(bm3)

    out = _head(acc2, hp, deg2,
                b1.reshape(1, -1), Wm1, bm1.reshape(1, -1),
                Wm2, bm2.reshape(1, -1), wm3p, bm3p.reshape(1, -1))
    return out[:, :NUM_CLASSES]


# speedup vs baseline: 11.9708x; 1.1473x over previous
import functools

import jax
import jax.numpy as jnp
from jax import lax
from jax.experimental import pallas as pl
from jax.experimental.pallas import tpu as pltpu
from jax.experimental.pallas import tpu_sc as plsc

N_NODES = 10000
N_EDGES = 320000
D_FEAT = 128
HIDDEN = 128
NUM_CLASSES = 10

NC = 2
NS = 16
CHUNK = 128
N_PAD = 10240
E_PAD = 327680
ROWS_PER_TILE = N_PAD // NS

_mesh = plsc.VectorSubcoreMesh(core_axis_name="c", subcore_axis_name="s")


N_CHUNKS = E_PAD // (NC * NS) // CHUNK


def _deg_body(dst_hbm, zeros_hbm, ones_hbm, out_hbm, acc, idxd, ones_v, dsems):
    c = lax.axis_index("c")
    s = lax.axis_index("s")
    r0 = s * ROWS_PER_TILE
    pltpu.sync_copy(zeros_hbm.at[pl.ds(r0, ROWS_PER_TILE)],
                    acc.at[pl.ds(r0, ROWS_PER_TILE)])
    pltpu.sync_copy(ones_hbm, ones_v)
    tile = s * NC + c
    e0 = tile * (N_CHUNKS * CHUNK)

    def dload(k, b):
        pltpu.async_copy(dst_hbm.at[pl.ds(e0 + k * CHUNK, CHUNK)], idxd[b],
                         dsems[b])

    def dwait(k, b):
        pltpu.make_async_copy(dst_hbm.at[pl.ds(e0 + k * CHUNK, CHUNK)],
                              idxd[b], dsems[b]).wait()

    plsc.subcore_barrier()

    def step(i, carry):
        pltpu.sync_copy(dst_hbm.at[pl.ds(e0 + i * CHUNK, CHUNK)], idxd[0])
        pltpu.sync_copy(ones_v, acc.at[idxd[0]], add=True)
        return carry

    lax.fori_loop(0, N_CHUNKS, step, 0)
    plsc.subcore_barrier()
    pltpu.sync_copy(acc.at[pl.ds(r0, ROWS_PER_TILE)],
                    out_hbm.at[c, pl.ds(r0, ROWS_PER_TILE)])


_deg_kernel = functools.partial(
    pl.kernel,
    out_type=jax.ShapeDtypeStruct((NC, N_PAD, 16), jnp.float32),
    mesh=_mesh,
    scratch_types=[
        pltpu.VMEM_SHARED((N_PAD, 16), jnp.float32),
        [pltpu.VMEM((CHUNK,), jnp.int32) for _ in range(2)],
        pltpu.VMEM((CHUNK, 16), jnp.float32),
        [pltpu.SemaphoreType.DMA for _ in range(2)],
    ],
)(_deg_body)


NBUF = 2


def _agg_body(src_hbm, dst_hbm, hp_hbm, zeros_hbm, out_hbm,
              acc, idxs, idxd, rows, gsems, ssems, dsems):
    c = lax.axis_index("c")
    s = lax.axis_index("s")
    r0 = s * ROWS_PER_TILE
    pltpu.sync_copy(zeros_hbm.at[pl.ds(r0, ROWS_PER_TILE)],
                    acc.at[pl.ds(r0, ROWS_PER_TILE)])
    tile = s * NC + c
    e0 = tile * (N_CHUNKS * CHUNK)
    plsc.subcore_barrier()

    def sload(k, b):
        pltpu.async_copy(src_hbm.at[pl.ds(e0 + k * CHUNK, CHUNK)], idxs[b],
                         ssems[b])

    def swait(k, b):
        pltpu.make_async_copy(src_hbm.at[pl.ds(e0 + k * CHUNK, CHUNK)],
                              idxs[b], ssems[b]).wait()

    def dload(k, b):
        pltpu.async_copy(dst_hbm.at[pl.ds(e0 + k * CHUNK, CHUNK)], idxd[b],
                         dsems[b])

    def dwait(k, b):
        pltpu.make_async_copy(dst_hbm.at[pl.ds(e0 + k * CHUNK, CHUNK)],
                              idxd[b], dsems[b]).wait()

    def gather(k, b):
        return pltpu.async_copy(hp_hbm.at[idxs[b]], rows[b], gsems[b])

    def gather_wait(k, b):
        pltpu.make_async_copy(hp_hbm.at[idxs[b]], rows[b], gsems[b]).wait()

    for b in range(NBUF):
        sload(b, b)
        dload(b, b)
    for b in range(NBUF):
        swait(b, b)
        gather(b, b)

    def step(i, carry):
        k0 = NBUF * i
        for b in range(NBUF):
            gather_wait(k0 + b, b)
            dwait(k0 + b, b)
            pltpu.sync_copy(rows[b], acc.at[idxd[b]], add=True)
            sload(k0 + NBUF + b, b)
            dload(k0 + NBUF + b, b)
        for b in range(NBUF):
            swait(k0 + NBUF + b, b)
            gather(k0 + NBUF + b, b)
        return carry

    lax.fori_loop(0, N_CHUNKS // NBUF - 1, step, 0)
    k0 = N_CHUNKS - NBUF
    for b in range(NBUF):
        gather_wait(k0 + b, b)
        dwait(k0 + b, b)
        pltpu.sync_copy(rows[b], acc.at[idxd[b]], add=True)
    plsc.subcore_barrier()
    pltpu.sync_copy(acc.at[pl.ds(r0, ROWS_PER_TILE)],
                    out_hbm.at[c, pl.ds(r0, ROWS_PER_TILE)])


_agg_kernel = functools.partial(
    pl.kernel,
    out_type=jax.ShapeDtypeStruct((NC, N_PAD, D_FEAT), jnp.float32),
    mesh=_mesh,
    scratch_types=[
        pltpu.VMEM_SHARED((N_PAD, D_FEAT), jnp.float32),
        [pltpu.VMEM((CHUNK,), jnp.int32) for _ in range(NBUF)],
        [pltpu.VMEM((CHUNK,), jnp.int32) for _ in range(NBUF)],
        [pltpu.VMEM((CHUNK, D_FEAT), jnp.float32) for _ in range(NBUF)],
        [pltpu.SemaphoreType.DMA for _ in range(NBUF)],
        [pltpu.SemaphoreType.DMA for _ in range(NBUF)],
        [pltpu.SemaphoreType.DMA for _ in range(NBUF)],
    ],
)(_agg_body)


ROW_BLK = 400


def _hprime_body(x_ref, w_ref, deg_ref, hp_ref):
    d = deg_ref[0, :, 0:1] + deg_ref[1, :, 0:1] + 1.0
    dis = lax.rsqrt(d)
    h = jnp.dot(x_ref[...], w_ref[...], preferred_element_type=jnp.float32)
    hp_ref[...] = dis * h


def _hprime(x, w1, deg2):
    return pl.pallas_call(
        _hprime_body,
        grid=(N_NODES // ROW_BLK,),
        in_specs=[
            pl.BlockSpec((ROW_BLK, D_FEAT), lambda r: (r, 0)),
            pl.BlockSpec((D_FEAT, HIDDEN), lambda r: (0, 0)),
            pl.BlockSpec((NC, ROW_BLK, 16), lambda r: (0, r, 0)),
        ],
        out_specs=pl.BlockSpec((ROW_BLK, HIDDEN), lambda r: (r, 0)),
        out_shape=jax.ShapeDtypeStruct((N_NODES, HIDDEN), jnp.float32),
    )(x, w1, deg2)


NC_PAD = 128


def _head_body(acc_ref, hp_ref, deg_ref, b1_ref, wm1_ref, bm1_ref,
               wm2_ref, bm2_ref, wm3_ref, bm3_ref, out_ref):
    d = deg_ref[0, :, 0:1] + deg_ref[1, :, 0:1] + 1.0
    dis = lax.rsqrt(d)
    accf = acc_ref[0] + acc_ref[1]
    g = dis * (accf + hp_ref[...]) + b1_ref[...]
    h1 = jnp.maximum(g, 0.0)
    h2 = jnp.maximum(
        jnp.dot(h1, wm1_ref[...], preferred_element_type=jnp.float32)
        + bm1_ref[...], 0.0)
    h3 = jnp.maximum(
        jnp.dot(h2, wm2_ref[...], preferred_element_type=jnp.float32)
        + bm2_ref[...], 0.0)
    logits = (jnp.dot(h3, wm3_ref[...], preferred_element_type=jnp.float32)
              + bm3_ref[...])
    m = jnp.max(logits, axis=1, keepdims=True)
    lse = jnp.log(jnp.sum(jnp.exp(logits - m), axis=1, keepdims=True)) + m
    out_ref[...] = logits - lse


def _head(acc2, hp, deg2, b1, wm1, bm1, wm2, bm2, wm3p, bm3p):
    full = lambda shape: pl.BlockSpec(shape, lambda r: tuple(0 for _ in shape))
    return pl.pallas_call(
        _head_body,
        grid=(N_NODES // ROW_BLK,),
        in_specs=[
            pl.BlockSpec((NC, ROW_BLK, D_FEAT), lambda r: (0, r, 0)),
            pl.BlockSpec((ROW_BLK, HIDDEN), lambda r: (r, 0)),
            pl.BlockSpec((NC, ROW_BLK, 16), lambda r: (0, r, 0)),
            full((1, HIDDEN)),
            full((HIDDEN, HIDDEN // 2)),
            full((1, HIDDEN // 2)),
            full((HIDDEN // 2, HIDDEN // 4)),
            full((1, HIDDEN // 4)),
            full((HIDDEN // 4, NC_PAD)),
            full((1, NC_PAD)),
        ],
        out_specs=pl.BlockSpec((ROW_BLK, NC_PAD), lambda r: (r, 0)),
        out_shape=jax.ShapeDtypeStruct((N_NODES, NC_PAD), jnp.float32),
    )(acc2, hp, deg2, b1, wm1, bm1, wm2, bm2, wm3p, bm3p)


@jax.jit
def kernel(x, edge_index, W1, b1, Wm1, bm1, Wm2, bm2, Wm3, bm3):
    src = edge_index[0]
    dst = edge_index[1]
    pad = E_PAD - N_EDGES
    src_p = jnp.concatenate([src, jnp.zeros((pad,), jnp.int32)])
    dst_p = jnp.concatenate([dst, jnp.full((pad,), N_NODES, jnp.int32)])

    zeros16 = jnp.zeros((N_PAD, 16), jnp.float32)
    ones16 = jnp.ones((CHUNK, 16), jnp.float32)
    zeros128 = jnp.zeros((N_PAD, D_FEAT), jnp.float32)

    deg2 = _deg_kernel(dst_p, zeros16, ones16)
    hp = _hprime(x, W1, deg2)
    acc2 = _agg_kernel(src_p, dst_p, hp, zeros128)

    wm3p = jnp.zeros((HIDDEN // 4, NC_PAD), jnp.float32).at[:, :NUM_CLASSES].set(Wm3)
    bm3p = jnp.full((NC_PAD,), -1e30, jnp.float32).at[:NUM_CLASSES].set(bm3)

    out = _head(acc2, hp, deg2,
                b1.reshape(1, -1), Wm1, bm1.reshape(1, -1),
                Wm2, bm2.reshape(1, -1), wm3p, bm3p.reshape(1, -1))
    return out[:, :NUM_CLASSES]
